# baseline (speedup 1.0000x reference)
import functools

import jax
import jax.numpy as jnp
from jax import lax
from jax.experimental import pallas as pl
from jax.experimental.pallas import tpu as pltpu
from jax.experimental.pallas import tpu_sc as plsc

KNBR = 3


def _knn_body(xrow_ref, xfull_ref, idx_ref, xt_ref, colsq_ref):
    xr = xrow_ref[...]
    xf = xfull_ref[...]
    R = xr.shape[1]
    N = xf.shape[1]

    @pl.when(pl.program_id(0) == 0)
    def _():
        colsq_ref[...] = jnp.sum(xf * xf, axis=0, keepdims=True)

    cross = lax.dot_general(-2.0 * xr, xf, (((0,), (0,)), ((), ())),
                            preferred_element_type=jnp.float32)
    d = colsq_ref[...] + cross
    iota = lax.broadcasted_iota(jnp.int32, (R, N), 1).astype(jnp.float32)
    BIG = jnp.float32(1e30)
    for k in range(KNBR):
        m = jnp.min(d, axis=1, keepdims=True)
        cand = jnp.where(d == m, iota, BIG)
        a = jnp.min(cand, axis=1, keepdims=True)
        idx_ref[k, 0] = a.astype(jnp.int32).T
        if k + 1 < KNBR:
            d = jnp.where(cand == a, BIG, d)
    xt_ref[...] = xr.T


def _knn_topk(xb, R):
    C, N = xb.shape
    NB = N // R
    idx, xt = pl.pallas_call(
        _knn_body,
        grid=(NB,),
        in_specs=[
            pl.BlockSpec((C, R), lambda j: (0, j)),
            pl.BlockSpec((C, N), lambda j: (0, 0)),
        ],
        out_specs=[
            pl.BlockSpec((KNBR, 1, 1, R), lambda j: (0, j, 0, 0)),
            pl.BlockSpec((R, C), lambda j: (j, 0)),
        ],
        out_shape=[
            jax.ShapeDtypeStruct((KNBR, NB, 1, R), jnp.int32),
            jax.ShapeDtypeStruct((N, C), jnp.float32),
        ],
        scratch_shapes=[pltpu.VMEM((1, N), jnp.float32)],
    )(xb, xb)
    return idx.reshape(KNBR, N), xt


def _gather_stage(xt, gid):
    N, C = xt.shape
    info = plsc.get_sparse_core_info()
    NW = info.num_cores * info.num_subcores
    CH = N // NW
    gid3 = gid.reshape(KNBR, NW, CH)
    mesh = plsc.VectorSubcoreMesh(core_axis_name="c", subcore_axis_name="s")

    @functools.partial(
        pl.kernel,
        out_type=jax.ShapeDtypeStruct((KNBR, N, C), jnp.float32),
        mesh=mesh,
        scratch_types=[
            pltpu.VMEM((KNBR, 1, CH), jnp.int32),
            pltpu.VMEM((KNBR, CH, C), jnp.float32),
            pltpu.SemaphoreType.DMA,
            pltpu.SemaphoreType.DMA,
        ],
    )
    def gather_k(xt_hbm, gid_hbm, out_hbm, idx_v, rows_v, gsem, ssem):
        wid = lax.axis_index("s") * info.num_cores + lax.axis_index("c")
        base = wid * CH
        for k in range(KNBR):
            pltpu.sync_copy(gid_hbm.at[k, pl.ds(wid, 1)], idx_v.at[k])
        cps = [
            pltpu.async_copy(xt_hbm.at[idx_v.at[k, 0]], rows_v.at[k], gsem)
            for k in range(KNBR)
        ]
        scps = []
        for k in range(KNBR):
            cps[k].wait()
            scps.append(pltpu.async_copy(rows_v.at[k],
                                         out_hbm.at[k, pl.ds(base, CH)], ssem))
        for scp in scps:
            scp.wait()

    return gather_k(xt, gid3)


def _conv_body(g_ref, w_ref, bias_ref, out_ref):
    OUT = w_ref.shape[1]
    R2 = g_ref.shape[1]
    acc = jnp.broadcast_to(bias_ref[...].T, (OUT, R2))
    for k in range(KNBR):
        acc = acc + lax.dot_general(w_ref[k], g_ref[k],
                                    (((1,), (1,)), ((), ())),
                                    preferred_element_type=jnp.float32)
    out_ref[...] = acc


def _conv_stage(prime, w3, bias, N, R2):
    OUT, C = w3.shape[1], w3.shape[2]
    NB = N // R2
    return pl.pallas_call(
        _conv_body,
        grid=(NB,),
        in_specs=[
            pl.BlockSpec((KNBR, R2, C), lambda j: (0, j, 0)),
            pl.BlockSpec((KNBR, OUT, C), lambda j: (0, 0, 0)),
            pl.BlockSpec((1, OUT), lambda j: (0, 0)),
        ],
        out_specs=pl.BlockSpec((OUT, R2), lambda j: (0, j)),
        out_shape=jax.ShapeDtypeStruct((OUT, N), jnp.float32),
    )(prime, w3, bias)


def kernel(x, W, b):
    B, C, N = x.shape
    OUT = W.shape[0]
    w3 = jnp.transpose(W, (2, 0, 1))
    brow = b.reshape(1, OUT)
    outs = []
    for bi in range(B):
        idx, xt = _knn_topk(x[bi], R=256)
        prime = _gather_stage(xt, idx)
        outs.append(_conv_stage(prime, w3, brow, N, R2=512))
    return jnp.stack(outs, axis=0)


# speedup vs baseline: 1.0647x; 1.0647x over previous
import functools

import jax
import jax.numpy as jnp
from jax import lax
from jax.experimental import pallas as pl
from jax.experimental.pallas import tpu as pltpu
from jax.experimental.pallas import tpu_sc as plsc

KNBR = 3


def _knn_body(xrow_ref, xfull_ref, idx_ref, xt_ref, colsq_ref):
    xr = xrow_ref[0]
    xf = xfull_ref[0]
    R = xr.shape[1]
    N = xf.shape[1]

    @pl.when(pl.program_id(1) == 0)
    def _():
        colsq_ref[...] = jnp.sum(xf * xf, axis=0, keepdims=True)

    cross = lax.dot_general(-2.0 * xr, xf, (((0,), (0,)), ((), ())),
                            preferred_element_type=jnp.float32)
    d = colsq_ref[...] + cross
    base = pl.program_id(0) * N
    iota = lax.broadcasted_iota(jnp.int32, (R, N), 1).astype(jnp.float32)
    BIG = jnp.float32(1e30)
    for k in range(KNBR):
        m = jnp.min(d, axis=1, keepdims=True)
        cand = jnp.where(d == m, iota, BIG)
        a = jnp.min(cand, axis=1, keepdims=True)
        idx_ref[k, 0, 0] = a.astype(jnp.int32).T + base
        if k + 1 < KNBR:
            d = jnp.where(cand == a, BIG, d)
    xt_ref[0] = xr.T


def _knn_topk(xg, R):
    G, C, N = xg.shape
    NB = N // R
    idx, xt = pl.pallas_call(
        _knn_body,
        grid=(G, NB),
        in_specs=[
            pl.BlockSpec((1, C, R), lambda b, j: (b, 0, j)),
            pl.BlockSpec((1, C, N), lambda b, j: (b, 0, 0)),
        ],
        out_specs=[
            pl.BlockSpec((KNBR, 1, 1, 1, R), lambda b, j: (0, b, j, 0, 0)),
            pl.BlockSpec((1, R, C), lambda b, j: (b, j, 0)),
        ],
        out_shape=[
            jax.ShapeDtypeStruct((KNBR, G, NB, 1, R), jnp.int32),
            jax.ShapeDtypeStruct((G, N, C), jnp.float32),
        ],
        scratch_shapes=[pltpu.VMEM((1, N), jnp.float32)],
    )(xg, xg)
    return idx.reshape(KNBR, G * N), xt.reshape(G * N, C)


def _gather_stage(xt, gid):
    N, C = xt.shape
    info = plsc.get_sparse_core_info()
    NW = info.num_cores * info.num_subcores
    CH = N // NW
    gid3 = gid.reshape(KNBR, NW, CH)
    mesh = plsc.VectorSubcoreMesh(core_axis_name="c", subcore_axis_name="s")

    @functools.partial(
        pl.kernel,
        out_type=jax.ShapeDtypeStruct((KNBR, N, C), jnp.float32),
        mesh=mesh,
        scratch_types=[
            pltpu.VMEM((KNBR, 1, CH), jnp.int32),
            pltpu.VMEM((KNBR, CH, C), jnp.float32),
            pltpu.SemaphoreType.DMA,
            pltpu.SemaphoreType.DMA,
        ],
    )
    def gather_k(xt_hbm, gid_hbm, out_hbm, idx_v, rows_v, gsem, ssem):
        wid = lax.axis_index("s") * info.num_cores + lax.axis_index("c")
        base = wid * CH
        for k in range(KNBR):
            pltpu.sync_copy(gid_hbm.at[k, pl.ds(wid, 1)], idx_v.at[k])
        cps = [
            pltpu.async_copy(xt_hbm.at[idx_v.at[k, 0]], rows_v.at[k], gsem)
            for k in range(KNBR)
        ]
        scps = []
        for k in range(KNBR):
            cps[k].wait()
            scps.append(pltpu.async_copy(rows_v.at[k],
                                         out_hbm.at[k, pl.ds(base, CH)], ssem))
        for scp in scps:
            scp.wait()

    return gather_k(xt, gid3)


def _conv_body(g_ref, w_ref, bias_ref, out_ref):
    OUT = w_ref.shape[1]
    R2 = g_ref.shape[2]
    acc = jnp.broadcast_to(bias_ref[...].T, (OUT, R2))
    for k in range(KNBR):
        acc = acc + lax.dot_general(w_ref[k], g_ref[k, 0],
                                    (((1,), (1,)), ((), ())),
                                    preferred_element_type=jnp.float32)
    out_ref[0] = acc


def _conv_stage(prime, w3, bias, G, N, R2):
    OUT, C = w3.shape[1], w3.shape[2]
    NB = N // R2
    return pl.pallas_call(
        _conv_body,
        grid=(G, NB),
        in_specs=[
            pl.BlockSpec((KNBR, 1, R2, C), lambda b, j: (0, b, j, 0)),
            pl.BlockSpec((KNBR, OUT, C), lambda b, j: (0, 0, 0)),
            pl.BlockSpec((1, OUT), lambda b, j: (0, 0)),
        ],
        out_specs=pl.BlockSpec((1, OUT, R2), lambda b, j: (b, 0, j)),
        out_shape=jax.ShapeDtypeStruct((G, OUT, N), jnp.float32),
    )(prime.reshape(KNBR, G, N, C), w3, bias)


def kernel(x, W, b):
    B, C, N = x.shape
    OUT = W.shape[0]
    G = 2
    w3 = jnp.transpose(W, (2, 0, 1))
    brow = b.reshape(1, OUT)
    outs = []
    for g in range(0, B, G):
        idx, xt = _knn_topk(x[g:g + G], R=256)
        prime = _gather_stage(xt, idx)
        outs.append(_conv_stage(prime, w3, brow, G, N, R2=512))
    return jnp.concatenate(outs, axis=0)


# speedup vs baseline: 1.2123x; 1.1386x over previous
import functools

import jax
import jax.numpy as jnp
from jax import lax
from jax.experimental import pallas as pl
from jax.experimental.pallas import tpu as pltpu
from jax.experimental.pallas import tpu_sc as plsc

KNBR = 3


def _knn_body(xrow_ref, xfull_ref, idx_ref, xt_ref, colsq_ref):
    xr = xrow_ref[0]
    xf = xfull_ref[0]
    R = xr.shape[1]
    N = xf.shape[1]

    @pl.when(pl.program_id(1) == 0)
    def _():
        colsq_ref[...] = jnp.sum(xf * xf, axis=0, keepdims=True)

    cross = lax.dot_general(-2.0 * xr, xf, (((0,), (0,)), ((), ())),
                            preferred_element_type=jnp.float32)
    d = colsq_ref[...] + cross
    base = pl.program_id(0) * N
    BIG = jnp.float32(1e30)
    for k in range(KNBR):
        iota = lax.broadcasted_iota(jnp.int32, (R, N), 1).astype(jnp.float32)
        m = jnp.min(d, axis=1, keepdims=True)
        cand = jnp.where(d == m, iota, BIG)
        a = jnp.min(cand, axis=1, keepdims=True)
        idx_ref[k, 0, 0] = a.astype(jnp.int32).T + base
        if k + 1 < KNBR:
            d = jnp.where(cand == a, BIG, d)
    xt_ref[0] = xr.T


def _knn_topk(xg, R):
    G, C, N = xg.shape
    NB = N // R
    idx, xt = pl.pallas_call(
        _knn_body,
        grid=(G, NB),
        in_specs=[
            pl.BlockSpec((1, C, R), lambda b, j: (b, 0, j)),
            pl.BlockSpec((1, C, N), lambda b, j: (b, 0, 0)),
        ],
        out_specs=[
            pl.BlockSpec((KNBR, 1, 1, 1, R), lambda b, j: (0, b, j, 0, 0)),
            pl.BlockSpec((1, R, C), lambda b, j: (b, j, 0)),
        ],
        out_shape=[
            jax.ShapeDtypeStruct((KNBR, G, NB, 1, R), jnp.int32),
            jax.ShapeDtypeStruct((G, N, C), jnp.float32),
        ],
        scratch_shapes=[pltpu.VMEM((1, N), jnp.float32)],
    )(xg, xg)
    return idx.reshape(KNBR, G * N), xt.reshape(G * N, C)


def _gather_stage(xt, gid):
    N, C = xt.shape
    info = plsc.get_sparse_core_info()
    NW = info.num_cores * info.num_subcores
    CH = N // NW
    gid3 = gid.reshape(KNBR, NW, CH)
    mesh = plsc.VectorSubcoreMesh(core_axis_name="c", subcore_axis_name="s")

    @functools.partial(
        pl.kernel,
        out_type=jax.ShapeDtypeStruct((KNBR, N, C), jnp.float32),
        mesh=mesh,
        scratch_types=[
            pltpu.VMEM((KNBR, 1, CH), jnp.int32),
            pltpu.VMEM((KNBR, CH, C), jnp.float32),
            pltpu.SemaphoreType.DMA,
            pltpu.SemaphoreType.DMA,
        ],
    )
    def gather_k(xt_hbm, gid_hbm, out_hbm, idx_v, rows_v, gsem, ssem):
        wid = lax.axis_index("s") * info.num_cores + lax.axis_index("c")
        base = wid * CH
        for k in range(KNBR):
            pltpu.sync_copy(gid_hbm.at[k, pl.ds(wid, 1)], idx_v.at[k])
        cps = [
            pltpu.async_copy(xt_hbm.at[idx_v.at[k, 0]], rows_v.at[k], gsem)
            for k in range(KNBR)
        ]
        scps = []
        for k in range(KNBR):
            cps[k].wait()
            scps.append(pltpu.async_copy(rows_v.at[k],
                                         out_hbm.at[k, pl.ds(base, CH)], ssem))
        for scp in scps:
            scp.wait()

    return gather_k(xt, gid3)


def _conv_body(g_ref, w_ref, bias_ref, out_ref):
    OUT = w_ref.shape[1]
    R2 = g_ref.shape[2]
    acc = jnp.broadcast_to(bias_ref[...].T, (OUT, R2))
    for k in range(KNBR):
        acc = acc + lax.dot_general(w_ref[k], g_ref[k, 0],
                                    (((1,), (1,)), ((), ())),
                                    preferred_element_type=jnp.float32)
    out_ref[0] = acc


def _conv_stage(prime, w3, bias, G, N, R2):
    OUT, C = w3.shape[1], w3.shape[2]
    NB = N // R2
    return pl.pallas_call(
        _conv_body,
        grid=(G, NB),
        in_specs=[
            pl.BlockSpec((KNBR, 1, R2, C), lambda b, j: (0, b, j, 0)),
            pl.BlockSpec((KNBR, OUT, C), lambda b, j: (0, 0, 0)),
            pl.BlockSpec((1, OUT), lambda b, j: (0, 0)),
        ],
        out_specs=pl.BlockSpec((1, OUT, R2), lambda b, j: (b, 0, j)),
        out_shape=jax.ShapeDtypeStruct((G, OUT, N), jnp.float32),
    )(prime.reshape(KNBR, G, N, C), w3, bias)


def kernel(x, W, b):
    B, C, N = x.shape
    OUT = W.shape[0]
    G = 2
    w3 = jnp.transpose(W, (2, 0, 1))
    brow = b.reshape(1, OUT)
    outs = []
    for g in range(0, B, G):
        idx, xt = _knn_topk(x[g:g + G], R=1024)
        prime = _gather_stage(xt, idx)
        outs.append(_conv_stage(prime, w3, brow, G, N, R2=512))
    return jnp.concatenate(outs, axis=0)


# speedup vs baseline: 1.3349x; 1.1011x over previous
import functools

import jax
import jax.numpy as jnp
from jax import lax
from jax.experimental import pallas as pl
from jax.experimental.pallas import tpu as pltpu
from jax.experimental.pallas import tpu_sc as plsc

KNBR = 3


def _knn_body(xrow_ref, xfull_ref, idx_ref, xt_ref, colsq_ref):
    xr = xrow_ref[0]
    xf = xfull_ref[0]
    R = xr.shape[1]
    N = xf.shape[1]

    @pl.when(pl.program_id(1) == 0)
    def _():
        colsq_ref[...] = jnp.sum(xf * xf, axis=0, keepdims=True)

    cross = lax.dot_general(-2.0 * xr, xf, (((0,), (0,)), ((), ())),
                            preferred_element_type=jnp.float32)
    d = colsq_ref[...] + cross
    base = pl.program_id(0) * N
    BIG = jnp.float32(1e30)
    for k in range(KNBR):
        iota = lax.broadcasted_iota(jnp.int32, (R, N), 1).astype(jnp.float32)
        m = jnp.min(d, axis=1, keepdims=True)
        cand = jnp.where(d == m, iota, BIG)
        a = jnp.min(cand, axis=1, keepdims=True)
        idx_ref[k, 0, 0] = a.astype(jnp.int32).T + base
        if k + 1 < KNBR:
            d = jnp.where(cand == a, BIG, d)
    xt_ref[0] = xr.T


def _knn_topk(x, g0, G, R):
    B, C, N = x.shape
    NB = N // R
    idx, xt = pl.pallas_call(
        _knn_body,
        grid=(G, NB),
        in_specs=[
            pl.BlockSpec((1, C, R), lambda b, j: (g0 + b, 0, j)),
            pl.BlockSpec((1, C, N), lambda b, j: (g0 + b, 0, 0)),
        ],
        out_specs=[
            pl.BlockSpec((KNBR, 1, 1, 1, R), lambda b, j: (0, b, j, 0, 0)),
            pl.BlockSpec((1, R, C), lambda b, j: (b, j, 0)),
        ],
        out_shape=[
            jax.ShapeDtypeStruct((KNBR, G, NB, 1, R), jnp.int32),
            jax.ShapeDtypeStruct((G, N, C), jnp.float32),
        ],
        scratch_shapes=[pltpu.VMEM((1, N), jnp.float32)],
    )(x, x)
    return idx.reshape(KNBR, G * N), xt.reshape(G * N, C)


def _gather_stage(xt, gid):
    N, C = xt.shape
    info = plsc.get_sparse_core_info()
    NW = info.num_cores * info.num_subcores
    CH = N // NW
    gid3 = gid.reshape(KNBR, NW, CH)
    mesh = plsc.VectorSubcoreMesh(core_axis_name="c", subcore_axis_name="s")

    @functools.partial(
        pl.kernel,
        out_type=jax.ShapeDtypeStruct((KNBR, N, C), jnp.float32),
        mesh=mesh,
        scratch_types=[
            pltpu.VMEM((KNBR, 1, CH), jnp.int32),
            pltpu.VMEM((KNBR, CH, C), jnp.float32),
            pltpu.SemaphoreType.DMA,
            pltpu.SemaphoreType.DMA,
        ],
    )
    def gather_k(xt_hbm, gid_hbm, out_hbm, idx_v, rows_v, gsem, ssem):
        wid = lax.axis_index("s") * info.num_cores + lax.axis_index("c")
        base = wid * CH
        for k in range(KNBR):
            pltpu.sync_copy(gid_hbm.at[k, pl.ds(wid, 1)], idx_v.at[k])
        cps = [
            pltpu.async_copy(xt_hbm.at[idx_v.at[k, 0]], rows_v.at[k], gsem)
            for k in range(KNBR)
        ]
        scps = []
        for k in range(KNBR):
            cps[k].wait()
            scps.append(pltpu.async_copy(rows_v.at[k],
                                         out_hbm.at[k, pl.ds(base, CH)], ssem))
        for scp in scps:
            scp.wait()

    return gather_k(xt, gid3)


def _conv_body_first(g_ref, w_ref, bias_ref, out_ref):
    _conv_accum(g_ref, w_ref, bias_ref, out_ref)


def _conv_body_next(g_ref, w_ref, bias_ref, acc_ref, out_ref):
    del acc_ref
    _conv_accum(g_ref, w_ref, bias_ref, out_ref)


def _conv_accum(g_ref, w_ref, bias_ref, out_ref):
    OUT = w_ref.shape[1]
    R2 = g_ref.shape[2]
    acc = jnp.broadcast_to(bias_ref[...].T, (OUT, R2))
    for k in range(KNBR):
        acc = acc + lax.dot_general(w_ref[k], g_ref[k, 0],
                                    (((1,), (1,)), ((), ())),
                                    preferred_element_type=jnp.float32)
    out_ref[0] = acc


def _conv_stage(prime, w3, bias, acc_out, B, g0, G, N, R2):
    OUT, C = w3.shape[1], w3.shape[2]
    NB = N // R2
    in_specs = [
        pl.BlockSpec((KNBR, 1, R2, C), lambda b, j: (0, b, j, 0)),
        pl.BlockSpec((KNBR, OUT, C), lambda b, j: (0, 0, 0)),
        pl.BlockSpec((1, OUT), lambda b, j: (0, 0)),
    ]
    args = [prime.reshape(KNBR, G, N, C), w3, bias]
    body = _conv_body_first
    aliases = {}
    if acc_out is not None:
        in_specs.append(pl.BlockSpec(memory_space=pl.ANY))
        args.append(acc_out)
        body = _conv_body_next
        aliases = {3: 0}
    return pl.pallas_call(
        body,
        grid=(G, NB),
        in_specs=in_specs,
        out_specs=pl.BlockSpec((1, OUT, R2), lambda b, j: (g0 + b, 0, j)),
        out_shape=jax.ShapeDtypeStruct((B, OUT, N), jnp.float32),
        input_output_aliases=aliases,
    )(*args)


def kernel(x, W, b):
    B, C, N = x.shape
    OUT = W.shape[0]
    G = 2
    w3 = jnp.transpose(W, (2, 0, 1))
    brow = b.reshape(1, OUT)
    out = None
    for g in range(0, B, G):
        idx, xt = _knn_topk(x, g, G, R=1024)
        prime = _gather_stage(xt, idx)
        out = _conv_stage(prime, w3, brow, out, B, g, G, N, R2=512)
    return out


# speedup vs baseline: 1.3723x; 1.0281x over previous
import functools

import jax
import jax.numpy as jnp
from jax import lax
from jax.experimental import pallas as pl
from jax.experimental.pallas import tpu as pltpu
from jax.experimental.pallas import tpu_sc as plsc

KNBR = 3


def _knn_body(xrow_ref, xfull_ref, idx_ref, xt_ref, colsq_ref):
    xr = xrow_ref[0]
    xf = xfull_ref[0]
    R = xr.shape[1]
    N = xf.shape[1]

    @pl.when(pl.program_id(1) == 0)
    def _():
        colsq_ref[...] = jnp.sum(xf * xf, axis=0, keepdims=True)

    cross = lax.dot_general(-2.0 * xr, xf, (((0,), (0,)), ((), ())),
                            preferred_element_type=jnp.float32)
    d = colsq_ref[...] + cross
    base = pl.program_id(0) * N
    BIG = jnp.float32(1e30)
    for k in range(KNBR):
        iota = lax.broadcasted_iota(jnp.int32, (R, N), 1).astype(jnp.float32)
        m = jnp.min(d, axis=1, keepdims=True)
        cand = jnp.where(d == m, iota, BIG)
        a = jnp.min(cand, axis=1, keepdims=True)
        idx_ref[k, 0, 0] = a.astype(jnp.int32).T + base
        if k + 1 < KNBR:
            d = jnp.where(cand == a, BIG, d)
    xt_ref[0] = xr.T


def _knn_topk(x, g0, G, R):
    B, C, N = x.shape
    NB = N // R
    idx, xt = pl.pallas_call(
        _knn_body,
        grid=(G, NB),
        in_specs=[
            pl.BlockSpec((1, C, R), lambda b, j: (g0 + b, 0, j)),
            pl.BlockSpec((1, C, N), lambda b, j: (g0 + b, 0, 0)),
        ],
        out_specs=[
            pl.BlockSpec((KNBR, 1, 1, 1, R), lambda b, j: (0, b, j, 0, 0)),
            pl.BlockSpec((1, R, C), lambda b, j: (b, j, 0)),
        ],
        out_shape=[
            jax.ShapeDtypeStruct((KNBR, G, NB, 1, R), jnp.int32),
            jax.ShapeDtypeStruct((G, N, C), jnp.float32),
        ],
        scratch_shapes=[pltpu.VMEM((1, N), jnp.float32)],
    )(x, x)
    return idx.reshape(KNBR, G * N), xt.reshape(G * N, C)


def _gather_stage(xt, gid):
    N, C = xt.shape
    info = plsc.get_sparse_core_info()
    NW = info.num_cores * info.num_subcores
    CH = N // NW
    gid3 = gid.reshape(KNBR, NW, CH)
    mesh = plsc.VectorSubcoreMesh(core_axis_name="c", subcore_axis_name="s")

    @functools.partial(
        pl.kernel,
        out_type=jax.ShapeDtypeStruct((KNBR, N, C), jnp.float32),
        mesh=mesh,
        scratch_types=[
            pltpu.VMEM((KNBR, 1, CH), jnp.int32),
            pltpu.VMEM((KNBR, CH, C), jnp.float32),
            pltpu.SemaphoreType.DMA,
            pltpu.SemaphoreType.DMA,
        ],
    )
    def gather_k(xt_hbm, gid_hbm, out_hbm, idx_v, rows_v, gsem, ssem):
        wid = lax.axis_index("s") * info.num_cores + lax.axis_index("c")
        base = wid * CH
        for k in range(KNBR):
            pltpu.sync_copy(gid_hbm.at[k, pl.ds(wid, 1)], idx_v.at[k])
        cps = [
            pltpu.async_copy(xt_hbm.at[idx_v.at[k, 0]], rows_v.at[k], gsem)
            for k in range(KNBR)
        ]
        scps = []
        for k in range(KNBR):
            cps[k].wait()
            scps.append(pltpu.async_copy(rows_v.at[k],
                                         out_hbm.at[k, pl.ds(base, CH)], ssem))
        for scp in scps:
            scp.wait()

    return gather_k(xt, gid3)


def _conv_body_first(g_ref, w_ref, bias_ref, out_ref):
    _conv_accum(g_ref, w_ref, bias_ref, out_ref)


def _conv_body_next(g_ref, w_ref, bias_ref, acc_ref, out_ref):
    del acc_ref
    _conv_accum(g_ref, w_ref, bias_ref, out_ref)


def _conv_accum(g_ref, w_ref, bias_ref, out_ref):
    OUT = w_ref.shape[1]
    R2 = g_ref.shape[2]
    acc = jnp.broadcast_to(bias_ref[...].T, (OUT, R2))
    for k in range(KNBR):
        acc = acc + lax.dot_general(w_ref[k], g_ref[k, 0],
                                    (((1,), (1,)), ((), ())),
                                    preferred_element_type=jnp.float32)
    out_ref[0] = acc


def _conv_stage(prime, w3, bias, acc_out, B, g0, G, N, R2):
    OUT, C = w3.shape[1], w3.shape[2]
    NB = N // R2
    in_specs = [
        pl.BlockSpec((KNBR, 1, R2, C), lambda b, j: (0, b, j, 0)),
        pl.BlockSpec((KNBR, OUT, C), lambda b, j: (0, 0, 0)),
        pl.BlockSpec((1, OUT), lambda b, j: (0, 0)),
    ]
    args = [prime.reshape(KNBR, G, N, C), w3, bias]
    body = _conv_body_first
    aliases = {}
    if acc_out is not None:
        in_specs.append(pl.BlockSpec(memory_space=pl.ANY))
        args.append(acc_out)
        body = _conv_body_next
        aliases = {3: 0}
    return pl.pallas_call(
        body,
        grid=(G, NB),
        in_specs=in_specs,
        out_specs=pl.BlockSpec((1, OUT, R2), lambda b, j: (g0 + b, 0, j)),
        out_shape=jax.ShapeDtypeStruct((B, OUT, N), jnp.float32),
        input_output_aliases=aliases,
    )(*args)


def kernel(x, W, b):
    B, C, N = x.shape
    OUT = W.shape[0]
    G = 2
    w3 = jnp.transpose(W, (2, 0, 1))
    brow = b.reshape(1, OUT)
    out = None
    for g in range(0, B, G):
        idx, xt = _knn_topk(x, g, G, R=1024)
        prime = _gather_stage(xt, idx)
        out = _conv_stage(prime, w3, brow, out, B, g, G, N, R2=1024)
    return out
